# baseline (speedup 1.0000x reference)
import jax
import jax.numpy as jnp
from jax.experimental import pallas as pl
from jax.experimental.pallas import tpu as pltpu

H, W, C, KH, KW = 32, 32, 32, 4, 4
B, XH, XW = 16, 16, 16
D = C * KH * KW
NQ = H * W
NPOS = B * XH * XW
NR = XH - KH + 1
NT = 4
TQ = NQ // NT


def _som_kernel(xt_ref, codes_ref, out_ref, p_ref, p2_ref, rmin_ref, ridx_ref):
    i = pl.program_id(0)

    @pl.when(i == 0)
    def _():
        parts = [
            xt_ref[pl.ds(kh * XW + kw, NPOS), :]
            for kh in range(KH)
            for kw in range(KW)
        ]
        p = jnp.concatenate(parts, axis=1)
        p_ref[...] = p
        p2_ref[...] = jnp.sum(p * p, axis=1, keepdims=True)
        rmin_ref[...] = jnp.full((NPOS, 1), jnp.inf, jnp.float32)
        ridx_ref[...] = jnp.zeros((NPOS, 1), jnp.int32)

    codes = codes_ref[...]
    c2 = jnp.sum(codes * codes, axis=0)
    cross = jax.lax.dot_general(
        p_ref[...], codes, (((1,), (0,)), ((), ())),
        preferred_element_type=jnp.float32)
    mse = (p2_ref[...] - 2.0 * cross + c2[None, :]) / D
    tmin = jnp.min(mse, axis=1, keepdims=True)
    lane = jax.lax.broadcasted_iota(jnp.int32, (NPOS, TQ), 1)
    tidx = jnp.min(jnp.where(mse == tmin, lane, NQ),
                   axis=1, keepdims=True) + i * TQ
    upd = tmin < rmin_ref[...]
    rmin_ref[...] = jnp.where(upd, tmin, rmin_ref[...])
    ridx_ref[...] = jnp.where(upd, tidx, ridx_ref[...])

    @pl.when(i == NT - 1)
    def _():
        idx = ridx_ref[...][:, 0]
        wr = (idx // W).astype(jnp.float32) / H
        wc = (idx % W).astype(jnp.float32) / W
        out_ref[...] = jnp.stack([wr, wc], axis=0)


@jax.jit
def kernel(x, som):
    xt = x.transpose(0, 2, 3, 1).reshape(NPOS, C)
    xt = jnp.pad(xt, ((0, KH * XW), (0, 0)))
    codes = som.transpose(3, 4, 2, 0, 1).reshape(D, NQ)
    out = pl.pallas_call(
        _som_kernel,
        grid=(NT,),
        in_specs=[
            pl.BlockSpec((NPOS + KH * XW, C), lambda i: (0, 0)),
            pl.BlockSpec((D, TQ), lambda i: (0, i)),
        ],
        out_specs=pl.BlockSpec((2, NPOS), lambda i: (0, 0)),
        out_shape=jax.ShapeDtypeStruct((2, NPOS), jnp.float32),
        scratch_shapes=[
            pltpu.VMEM((NPOS, D), jnp.float32),
            pltpu.VMEM((NPOS, 1), jnp.float32),
            pltpu.VMEM((NPOS, 1), jnp.float32),
            pltpu.VMEM((NPOS, 1), jnp.int32),
        ],
    )(xt, codes)
    out = out.reshape(2, B, XH, XW)[:, :, :NR, :NR]
    return out.transpose(1, 0, 2, 3)


# speedup vs baseline: 1.0458x; 1.0458x over previous
import jax
import jax.numpy as jnp
from jax.experimental import pallas as pl

H, W, C, KH, KW = 32, 32, 32, 4, 4
B, XH, XW = 16, 16, 16
D = C * KH * KW
NQ = H * W
NPOS = B * XH * XW
NR = XH - KH + 1
NT = 4
TQ = NQ // NT


def _som_kernel(xt_ref, codes_ref, out_ref):
    parts = [
        xt_ref[pl.ds(kh * XW + kw, NPOS), :]
        for kh in range(KH)
        for kw in range(KW)
    ]
    p = jnp.concatenate(parts, axis=1)
    p2 = jnp.sum(p * p, axis=1, keepdims=True)
    lane = jax.lax.broadcasted_iota(jnp.int32, (NPOS, TQ), 1)
    rmin = jnp.full((NPOS, 1), jnp.inf, jnp.float32)
    ridx = jnp.zeros((NPOS, 1), jnp.int32)
    for t in range(NT):
        cb = codes_ref[:, t * TQ:(t + 1) * TQ]
        c2 = jnp.sum(cb * cb, axis=0)
        cross = jax.lax.dot_general(
            p, cb, (((1,), (0,)), ((), ())),
            preferred_element_type=jnp.float32)
        mse = (p2 - 2.0 * cross + c2[None, :]) / D
        tmin = jnp.min(mse, axis=1, keepdims=True)
        tidx = jnp.min(jnp.where(mse == tmin, lane, NQ),
                       axis=1, keepdims=True) + t * TQ
        upd = tmin < rmin
        rmin = jnp.where(upd, tmin, rmin)
        ridx = jnp.where(upd, tidx, ridx)
    idx = ridx[:, 0]
    wr = (idx // W).astype(jnp.float32) / H
    wc = (idx % W).astype(jnp.float32) / W
    out_ref[...] = jnp.stack([wr, wc], axis=0)


@jax.jit
def kernel(x, som):
    xt = x.transpose(0, 2, 3, 1).reshape(NPOS, C)
    xt = jnp.pad(xt, ((0, KH * XW), (0, 0)))
    codes = som.transpose(3, 4, 2, 0, 1).reshape(D, NQ)
    out = pl.pallas_call(
        _som_kernel,
        out_shape=jax.ShapeDtypeStruct((2, NPOS), jnp.float32),
    )(xt, codes)
    out = out.reshape(2, B, XH, XW)[:, :, :NR, :NR]
    return out.transpose(1, 0, 2, 3)


# speedup vs baseline: 1.2558x; 1.2008x over previous
import jax
import jax.numpy as jnp
from jax.experimental import pallas as pl

H, W, C, KH, KW = 32, 32, 32, 4, 4
B, XH, XW = 16, 16, 16
D = C * KH * KW
NQ = H * W
NPOS = B * XH * XW
NR = XH - KH + 1


def _som_kernel(xt_ref, codes_ref, out_ref):
    codes = codes_ref[...]
    c2 = jnp.sum(codes * codes, axis=1)
    parts = [
        xt_ref[pl.ds(kh * XW + kw, NPOS), :]
        for kh in range(KH)
        for kw in range(KW)
    ]
    p = jnp.concatenate(parts, axis=1)
    p2 = jnp.sum(p * p, axis=1, keepdims=True)
    cross = jax.lax.dot_general(
        p, codes, (((1,), (1,)), ((), ())),
        preferred_element_type=jnp.float32)
    mse = (p2 - 2.0 * cross + c2[None, :]) / D
    idx = jnp.argmin(mse, axis=1)
    wr = (idx // W).astype(jnp.float32) / H
    wc = (idx % W).astype(jnp.float32) / W
    out_ref[...] = jnp.stack([wr, wc], axis=0)


@jax.jit
def kernel(x, som):
    xt = x.transpose(0, 2, 3, 1).reshape(NPOS, C)
    xt = jnp.pad(xt, ((0, KH * XW), (0, 0)))
    codes = som.transpose(0, 1, 3, 4, 2).reshape(NQ, D)
    out = pl.pallas_call(
        _som_kernel,
        out_shape=jax.ShapeDtypeStruct((2, NPOS), jnp.float32),
    )(xt, codes)
    out = out.reshape(2, B, XH, XW)[:, :, :NR, :NR]
    return out.transpose(1, 0, 2, 3)


# speedup vs baseline: 1.2757x; 1.0159x over previous
import jax
import jax.numpy as jnp
from jax.experimental import pallas as pl

H, W, C, KH, KW = 32, 32, 32, 4, 4
B, XH, XW = 16, 16, 16
D = C * KH * KW
NQ = H * W
NPOS = B * XH * XW
NR = XH - KH + 1


def _som_kernel(xt_ref, codes_ref, out_ref):
    codes = codes_ref[...]
    c2 = jnp.sum(codes * codes, axis=1)
    parts = [
        xt_ref[pl.ds(kh * XW + kw, NPOS), :]
        for kh in range(KH)
        for kw in range(KW)
    ]
    p = jnp.concatenate(parts, axis=1)
    p2 = jnp.sum(p * p, axis=1, keepdims=True)
    cross = jax.lax.dot_general(
        p, codes, (((1,), (1,)), ((), ())),
        preferred_element_type=jnp.float32)
    mse = p2 - 2.0 * cross + c2[None, :]
    idx = jnp.argmin(mse, axis=1)
    wr = (idx // W).astype(jnp.float32) / H
    wc = (idx % W).astype(jnp.float32) / W
    out_ref[...] = jnp.stack([wr, wc], axis=0)


@jax.jit
def kernel(x, som):
    xt = x.transpose(0, 2, 3, 1).reshape(NPOS, C)
    xt = jnp.pad(xt, ((0, KH * XW), (0, 0)))
    codes = som.transpose(0, 1, 3, 4, 2).reshape(NQ, D)
    out = pl.pallas_call(
        _som_kernel,
        out_shape=jax.ShapeDtypeStruct((2, NPOS), jnp.float32),
    )(xt, codes)
    out = out.reshape(2, B, XH, XW)[:, :, :NR, :NR]
    return out.transpose(1, 0, 2, 3)
